# baseline (speedup 1.0000x reference)
import functools

import jax
import jax.numpy as jnp
from jax import lax
from jax.experimental import pallas as pl
from jax.experimental.pallas import tpu as pltpu
from jax.experimental.pallas import tpu_sc as plsc

_BATCH = 16384
_EMB = 16


@functools.lru_cache(maxsize=None)
def _build_mf_kernel():
    info = plsc.get_sparse_core_info()
    nc, ns, nl = info.num_cores, info.num_subcores, info.num_lanes
    nw = nc * ns
    bpw = _BATCH // nw
    nchunks = bpw // nl
    pitch = nl + 1
    mesh = plsc.VectorSubcoreMesh(core_axis_name="c", subcore_axis_name="s")

    @functools.partial(
        pl.kernel,
        mesh=mesh,
        out_type=jax.ShapeDtypeStruct((_BATCH,), jnp.float32),
        compiler_params=pltpu.CompilerParams(
            needs_layout_passes=False, use_tc_tiling_on_sc=False),
        scratch_types=[
            pltpu.VMEM((bpw,), jnp.int32),
            pltpu.VMEM((bpw,), jnp.int32),
            pltpu.VMEM((bpw, _EMB), jnp.float32),
            pltpu.VMEM((bpw, _EMB), jnp.float32),
            pltpu.VMEM((bpw,), jnp.float32),
            pltpu.VMEM((bpw,), jnp.float32),
            pltpu.VMEM((nl,), jnp.float32),
            pltpu.VMEM((_EMB * (nl + 1),), jnp.float32),
            pltpu.VMEM((bpw,), jnp.float32),
            pltpu.SemaphoreType.DMA,
        ],
    )
    def mf(u_id, i_id, uemb, ubias, iemb, ibias, mean16, out,
           uidx_v, iidx_v, urows_v, irows_v, ub_v, ib_v, mean_v, pt_v,
           out_v, sem):
        wid = lax.axis_index("s") * nc + lax.axis_index("c")
        base = wid * bpw
        pltpu.sync_copy(u_id.at[pl.ds(base, bpw)], uidx_v)
        pltpu.sync_copy(i_id.at[pl.ds(base, bpw)], iidx_v)
        pltpu.sync_copy(mean16, mean_v)
        c1 = pltpu.async_copy(uemb.at[uidx_v], urows_v, sem)
        c2 = pltpu.async_copy(iemb.at[iidx_v], irows_v, sem)
        c3 = pltpu.async_copy(ubias.at[uidx_v], ub_v, sem)
        c4 = pltpu.async_copy(ibias.at[iidx_v], ib_v, sem)
        c1.wait()
        c2.wait()
        c3.wait()
        c4.wait()

        col = lax.iota(jnp.int32, nl) * pitch
        mean_vec = mean_v[...]

        def chunk(c, carry):
            for l in range(nl):
                p = c * nl + l
                prod = urows_v[p, :] * irows_v[p, :]
                plsc.store_scatter(pt_v, [col + l], prod)
            acc = ub_v[pl.ds(c * nl, nl)] + ib_v[pl.ds(c * nl, nl)] + mean_vec
            for d in range(_EMB):
                acc = acc + pt_v[pl.ds(d * pitch, nl)]
            out_v[pl.ds(c * nl, nl)] = 1.0 / (1.0 + jnp.exp(-acc))
            return carry

        lax.fori_loop(0, nchunks, chunk, 0)
        pltpu.sync_copy(out_v, out.at[pl.ds(base, bpw)])

    return mf


def kernel(data, user_emb, user_bias, item_emb, item_bias, mean):
    u_id = data[0].astype(jnp.int32)
    i_id = data[1].astype(jnp.int32)
    mean16 = jnp.broadcast_to(mean.astype(jnp.float32), (16,))
    mf = _build_mf_kernel()
    return mf(u_id, i_id, user_emb, user_bias.reshape(-1),
              item_emb, item_bias.reshape(-1), mean16)
